# baseline (speedup 1.0000x reference)
import functools

import jax
import jax.numpy as jnp
from jax import lax
from jax.experimental import pallas as pl
from jax.experimental.pallas import tpu as pltpu
from jax.experimental.pallas import tpu_sc as plsc

N = 10000
D = 128
E = 320000

NC = 2
NS = 16
NW = NC * NS

NP = 10112
CHUNK = 80
AGG_CHUNKS = 125
EDGES_PER_TILE = AGG_CHUNKS * CHUNK
ROWS_PER_TILE = NP // NS

DG = 10240
DEG_CHUNK = 128
DEG_CHUNKS = 160
DEGPAD = NW * DEG_CHUNKS * DEG_CHUNK
DEG_PER_TILE = 2 * DG // NS
DEG_WINDOW = 16

NBUF = 4
NSI = 5

_MESH = plsc.VectorSubcoreMesh(
    core_axis_name="c", subcore_axis_name="s", num_cores=NC, num_subcores=NS)


def _deg_body(idx_hbm, out_hbm, idx_v, ones_v, zero_v, acc, dsem):
    cid = lax.axis_index("c")
    sid = lax.axis_index("s")
    wid = sid * NC + cid
    for k in range(DEG_CHUNK // 16):
        ones_v[pl.ds(k * 16, 16)] = jnp.ones((16,), jnp.float32)
        zero_v[pl.ds(k * 16, 16)] = jnp.zeros((16,), jnp.float32)

    def zero_acc(k, carry):
        pltpu.sync_copy(
            zero_v, acc.at[pl.ds(sid * DEG_PER_TILE + k * DEG_CHUNK, DEG_CHUNK)])
        return carry

    lax.fori_loop(0, DEG_PER_TILE // DEG_CHUNK, zero_acc, 0)
    pltpu.sync_copy(idx_hbm.at[wid], idx_v)
    plsc.subcore_barrier()

    def scat_issue(j):
        pltpu.async_copy(ones_v, acc.at[idx_v.at[j]], dsem, add=True)

    def scat_wait(j):
        pltpu.make_async_copy(ones_v, acc.at[idx_v.at[j]], dsem).wait()

    for j in range(DEG_WINDOW):
        scat_issue(j)

    def roll(j, carry):
        scat_issue(j + DEG_WINDOW)
        scat_wait(j)
        return carry

    lax.fori_loop(0, DEG_CHUNKS - DEG_WINDOW, roll, 0)

    def drain(j, carry):
        scat_wait(j)
        return carry

    lax.fori_loop(DEG_CHUNKS - DEG_WINDOW, DEG_CHUNKS, drain, 0)
    plsc.subcore_barrier()
    pltpu.sync_copy(acc.at[pl.ds(sid * DEG_PER_TILE, DEG_PER_TILE)],
                    out_hbm.at[cid, pl.ds(sid * DEG_PER_TILE, DEG_PER_TILE)])


_deg_kernel = functools.partial(
    pl.kernel,
    out_type=jax.ShapeDtypeStruct((NC, 2 * DG), jnp.float32),
    mesh=_MESH,
    scratch_types=[
        pltpu.VMEM((DEG_CHUNKS, DEG_CHUNK), jnp.int32),
        pltpu.VMEM((DEG_CHUNK,), jnp.float32),
        pltpu.VMEM((DEG_CHUNK,), jnp.float32),
        pltpu.VMEM_SHARED((2 * DG,), jnp.float32),
        pltpu.SemaphoreType.DMA,
    ],
)(_deg_body)


def _agg_body(hs_hbm, src_hbm, dst_hbm, out_hbm,
              buf0, buf1, buf2, buf3, si0, si1, si2, si3, si4, di0, di1, acc,
              gs0, gs1, gs2, gs3, ss0, ss1, ss2, ss3, ss4, ds0, ds1):
    cid = lax.axis_index("c")
    sid = lax.axis_index("s")
    wid = sid * NC + cid
    base = pl.multiple_of(wid * EDGES_PER_TILE, CHUNK)

    bufs = (buf0, buf1, buf2, buf3)
    gsems = (gs0, gs1, gs2, gs3)
    sidx = (si0, si1, si2, si3, si4)
    ssems = (ss0, ss1, ss2, ss3, ss4)
    didx = (di0, di1)
    dsems = (ds0, ds1)

    def zero_buf0(i, carry):
        for k in range(D // 16):
            buf0[i, pl.ds(k * 16, 16)] = jnp.zeros((16,), jnp.float32)
        return carry

    lax.fori_loop(0, CHUNK, zero_buf0, 0)
    for k in range(ROWS_PER_TILE // CHUNK):
        pltpu.sync_copy(buf0, acc.at[pl.ds(sid * ROWS_PER_TILE + k * CHUNK, CHUNK)])
    rem = ROWS_PER_TILE % CHUNK
    if rem:
        pltpu.sync_copy(
            buf0.at[pl.ds(0, rem)],
            acc.at[pl.ds(sid * ROWS_PER_TILE + (ROWS_PER_TILE - rem), rem)])
    plsc.subcore_barrier()

    def sload(c):
        return (src_hbm.at[pl.ds(base + c * CHUNK, CHUNK)],
                sidx[c % NSI], ssems[c % NSI])

    def dload(c):
        return (dst_hbm.at[pl.ds(base + c * CHUNK, CHUNK)],
                didx[c % 2], dsems[c % 2])

    for c in range(NSI):
        pltpu.sync_copy(src_hbm.at[pl.ds(base + c * CHUNK, CHUNK)], sidx[c])
    for c in range(2):
        pltpu.sync_copy(dst_hbm.at[pl.ds(base + c * CHUNK, CHUNK)], didx[c])
    for c in range(NBUF):
        pltpu.async_copy(hs_hbm.at[sidx[c]], bufs[c], gsems[c])

    for c in range(AGG_CHUNKS):
        b = c % NBUF
        pltpu.make_async_copy(hs_hbm.at[sidx[c % NSI]], bufs[b], gsems[b]).wait()
        if c >= 2:
            pltpu.make_async_copy(*dload(c)).wait()
        pltpu.sync_copy(bufs[b], acc.at[didx[c % 2]], add=True)
        if c + NSI < AGG_CHUNKS:
            pltpu.async_copy(*sload(c + NSI))
        if c + 2 < AGG_CHUNKS:
            pltpu.async_copy(*dload(c + 2))
        if c + NBUF < AGG_CHUNKS:
            if c + NBUF >= NSI:
                pltpu.make_async_copy(*sload(c + NBUF)).wait()
            pltpu.async_copy(hs_hbm.at[sidx[(c + NBUF) % NSI]], bufs[b], gsems[b])
    plsc.subcore_barrier()
    pltpu.sync_copy(acc.at[pl.ds(sid * ROWS_PER_TILE, ROWS_PER_TILE)],
                    out_hbm.at[cid, pl.ds(sid * ROWS_PER_TILE, ROWS_PER_TILE)])


_agg_kernel = functools.partial(
    pl.kernel,
    out_type=jax.ShapeDtypeStruct((NC, NP, D), jnp.float32),
    mesh=_MESH,
    scratch_types=[
        pltpu.VMEM((CHUNK, D), jnp.float32),
        pltpu.VMEM((CHUNK, D), jnp.float32),
        pltpu.VMEM((CHUNK, D), jnp.float32),
        pltpu.VMEM((CHUNK, D), jnp.float32),
        pltpu.VMEM((CHUNK,), jnp.int32),
        pltpu.VMEM((CHUNK,), jnp.int32),
        pltpu.VMEM((CHUNK,), jnp.int32),
        pltpu.VMEM((CHUNK,), jnp.int32),
        pltpu.VMEM((CHUNK,), jnp.int32),
        pltpu.VMEM((CHUNK,), jnp.int32),
        pltpu.VMEM((CHUNK,), jnp.int32),
        pltpu.VMEM_SHARED((NP, D), jnp.float32),
        pltpu.SemaphoreType.DMA,
        pltpu.SemaphoreType.DMA,
        pltpu.SemaphoreType.DMA,
        pltpu.SemaphoreType.DMA,
        pltpu.SemaphoreType.DMA,
        pltpu.SemaphoreType.DMA,
        pltpu.SemaphoreType.DMA,
        pltpu.SemaphoreType.DMA,
        pltpu.SemaphoreType.DMA,
        pltpu.SemaphoreType.DMA,
        pltpu.SemaphoreType.DMA,
    ],
)(_agg_body)


def _bn_scale_body(h_ref, gamma_ref, beta_ref, norm_ref, o_ref):
    h = h_ref[...]
    mean = jnp.mean(h, axis=0, keepdims=True)
    diff = h - mean
    var = jnp.mean(diff * diff, axis=0, keepdims=True)
    rstd = lax.rsqrt(var + 1e-5)
    o_ref[...] = (diff * (rstd * gamma_ref[...]) + beta_ref[...]) * norm_ref[...]


def _bn_scale(h, gamma, beta, norm_col):
    return pl.pallas_call(
        _bn_scale_body,
        out_shape=jax.ShapeDtypeStruct((N, D), jnp.float32),
    )(h, gamma.reshape(1, D), beta.reshape(1, D), norm_col)


def _mm_body(agg_ref, normin_ref, w_ref, b_ref, o_ref):
    m = (agg_ref[0, :N, :] + agg_ref[1, :N, :]) * normin_ref[...]
    mm = jnp.dot(m, w_ref[...], preferred_element_type=jnp.float32)
    o_ref[...] = jnp.maximum(mm + b_ref[...], 0.0)


def _mm_relu(agg, norm_in_col, w, b):
    return pl.pallas_call(
        _mm_body,
        out_shape=jax.ShapeDtypeStruct((N, D), jnp.float32),
    )(agg, norm_in_col, w, b.reshape(1, D))


def _mm_bn_body(agg_ref, normin_ref, w_ref, b_ref,
                gamma_ref, beta_ref, normout_ref, o_ref):
    m = (agg_ref[0, :N, :] + agg_ref[1, :N, :]) * normin_ref[...]
    mm = jnp.dot(m, w_ref[...], preferred_element_type=jnp.float32)
    h = jnp.maximum(mm + b_ref[...], 0.0)
    mean = jnp.mean(h, axis=0, keepdims=True)
    diff = h - mean
    var = jnp.mean(diff * diff, axis=0, keepdims=True)
    rstd = lax.rsqrt(var + 1e-5)
    o_ref[...] = (diff * (rstd * gamma_ref[...]) + beta_ref[...]) * normout_ref[...]


def _mm_relu_bn_scale(agg, norm_in_col, w, b, gamma, beta, norm_out_col):
    return pl.pallas_call(
        _mm_bn_body,
        out_shape=jax.ShapeDtypeStruct((N, D), jnp.float32),
    )(agg, norm_in_col, w, b.reshape(1, D),
      gamma.reshape(1, D), beta.reshape(1, D), norm_out_col)


def kernel(x, edge_index, gamma1, beta1, W1, b1, gamma2, beta2, W2, b2):
    src = edge_index[0].astype(jnp.int32)
    dst = edge_index[1].astype(jnp.int32)

    n_deg_pad = DEGPAD - 2 * E
    deg_pad = N + (jnp.arange(n_deg_pad, dtype=jnp.int32) % (DG - N))
    deg_idx = jnp.concatenate([src, dst + DG, deg_pad]).reshape(
        NW, DEG_CHUNKS, DEG_CHUNK)

    deg_parts = _deg_kernel(deg_idx)
    deg = (deg_parts[0] + deg_parts[1]).reshape(2, DG)
    norm_out = lax.rsqrt(jnp.maximum(deg[0, :N], 1.0))[:, None]
    norm_in = lax.rsqrt(jnp.maximum(deg[1, :N], 1.0))[:, None]

    hs1 = _bn_scale(x, gamma1, beta1, norm_out)
    agg1 = _agg_kernel(hs1, src, dst)
    hs2 = _mm_relu_bn_scale(agg1, norm_in, W1, b1, gamma2, beta2, norm_out)
    agg2 = _agg_kernel(hs2, src, dst)
    h2 = _mm_relu(agg2, norm_in, W2, b2)
    return h2


# speedup vs baseline: 1.0046x; 1.0046x over previous
import functools

import jax
import jax.numpy as jnp
from jax import lax
from jax.experimental import pallas as pl
from jax.experimental.pallas import tpu as pltpu
from jax.experimental.pallas import tpu_sc as plsc

N = 10000
D = 128
E = 320000

NC = 2
NS = 16
NW = NC * NS

NP = 10112
CHUNK = 128
AGG_CHUNKS = 80
EPAD = NW * AGG_CHUNKS * CHUNK
EDGES_PER_TILE = AGG_CHUNKS * CHUNK
ROWS_PER_TILE = NP // NS

DG = 10240
DEG_CHUNKS = 160
DEGPAD = NW * DEG_CHUNKS * CHUNK
DEG_PER_TILE = 2 * DG // NS
DEG_WINDOW = 16

NBUF = 3

_MESH = plsc.VectorSubcoreMesh(
    core_axis_name="c", subcore_axis_name="s", num_cores=NC, num_subcores=NS)


def _deg_body(idx_hbm, out_hbm, idx_v, ones_v, zero_v, acc, dsem):
    cid = lax.axis_index("c")
    sid = lax.axis_index("s")
    wid = sid * NC + cid
    for k in range(CHUNK // 16):
        ones_v[pl.ds(k * 16, 16)] = jnp.ones((16,), jnp.float32)
        zero_v[pl.ds(k * 16, 16)] = jnp.zeros((16,), jnp.float32)

    def zero_acc(k, carry):
        pltpu.sync_copy(zero_v, acc.at[pl.ds(sid * DEG_PER_TILE + k * CHUNK, CHUNK)])
        return carry

    lax.fori_loop(0, DEG_PER_TILE // CHUNK, zero_acc, 0)
    pltpu.sync_copy(idx_hbm.at[wid], idx_v)
    plsc.subcore_barrier()

    def scat_issue(j):
        pltpu.async_copy(ones_v, acc.at[idx_v.at[j]], dsem, add=True)

    def scat_wait(j):
        pltpu.make_async_copy(ones_v, acc.at[idx_v.at[j]], dsem).wait()

    for j in range(DEG_WINDOW):
        scat_issue(j)

    def roll(j, carry):
        scat_issue(j + DEG_WINDOW)
        scat_wait(j)
        return carry

    lax.fori_loop(0, DEG_CHUNKS - DEG_WINDOW, roll, 0)

    def drain(j, carry):
        scat_wait(j)
        return carry

    lax.fori_loop(DEG_CHUNKS - DEG_WINDOW, DEG_CHUNKS, drain, 0)
    plsc.subcore_barrier()
    pltpu.sync_copy(acc.at[pl.ds(sid * DEG_PER_TILE, DEG_PER_TILE)],
                    out_hbm.at[cid, pl.ds(sid * DEG_PER_TILE, DEG_PER_TILE)])


_deg_kernel = functools.partial(
    pl.kernel,
    out_type=jax.ShapeDtypeStruct((NC, 2 * DG), jnp.float32),
    mesh=_MESH,
    scratch_types=[
        pltpu.VMEM((DEG_CHUNKS, CHUNK), jnp.int32),
        pltpu.VMEM((CHUNK,), jnp.float32),
        pltpu.VMEM((CHUNK,), jnp.float32),
        pltpu.VMEM_SHARED((2 * DG,), jnp.float32),
        pltpu.SemaphoreType.DMA,
    ],
)(_deg_body)


def _agg_body(hs_hbm, src_hbm, dst_hbm, out_hbm,
              buf0, buf1, buf2, si0, si1, si2, si3, di0, di1, acc,
              gs0, gs1, gs2, ss0, ss1, ss2, ss3, ds0, ds1):
    cid = lax.axis_index("c")
    sid = lax.axis_index("s")
    wid = sid * NC + cid
    base = pl.multiple_of(wid * EDGES_PER_TILE, CHUNK)

    bufs = (buf0, buf1, buf2)
    gsems = (gs0, gs1, gs2)
    sidx = (si0, si1, si2, si3)
    ssems = (ss0, ss1, ss2, ss3)
    didx = (di0, di1)
    dsems = (ds0, ds1)

    def zero_buf0(i, carry):
        for k in range(D // 16):
            buf0[i, pl.ds(k * 16, 16)] = jnp.zeros((16,), jnp.float32)
        return carry

    lax.fori_loop(0, CHUNK, zero_buf0, 0)
    for k in range(ROWS_PER_TILE // CHUNK):
        pltpu.sync_copy(buf0, acc.at[pl.ds(sid * ROWS_PER_TILE + k * CHUNK, CHUNK)])
    rem = ROWS_PER_TILE % CHUNK
    if rem:
        pltpu.sync_copy(
            buf0.at[pl.ds(0, rem)],
            acc.at[pl.ds(sid * ROWS_PER_TILE + (ROWS_PER_TILE - rem), rem)])
    plsc.subcore_barrier()

    def sload(c):
        return src_hbm.at[pl.ds(base + c * CHUNK, CHUNK)], sidx[c % 4], ssems[c % 4]

    def dload(c):
        return dst_hbm.at[pl.ds(base + c * CHUNK, CHUNK)], didx[c % 2], dsems[c % 2]

    for c in range(4):
        pltpu.sync_copy(src_hbm.at[pl.ds(base + c * CHUNK, CHUNK)], sidx[c])
    for c in range(2):
        pltpu.sync_copy(dst_hbm.at[pl.ds(base + c * CHUNK, CHUNK)], didx[c])
    for c in range(NBUF):
        pltpu.async_copy(hs_hbm.at[sidx[c]], bufs[c], gsems[c])

    for c in range(AGG_CHUNKS):
        b = c % NBUF
        pltpu.make_async_copy(hs_hbm.at[sidx[c % 4]], bufs[b], gsems[b]).wait()
        if c >= 2:
            pltpu.make_async_copy(*dload(c)).wait()
        pltpu.sync_copy(bufs[b], acc.at[didx[c % 2]], add=True)
        if c + 4 < AGG_CHUNKS:
            pltpu.async_copy(*sload(c + 4))
        if c + 2 < AGG_CHUNKS:
            pltpu.async_copy(*dload(c + 2))
        if c + NBUF < AGG_CHUNKS:
            if c + NBUF >= 4:
                pltpu.make_async_copy(*sload(c + NBUF)).wait()
            pltpu.async_copy(hs_hbm.at[sidx[(c + NBUF) % 4]], bufs[b], gsems[b])
    plsc.subcore_barrier()
    pltpu.sync_copy(acc.at[pl.ds(sid * ROWS_PER_TILE, ROWS_PER_TILE)],
                    out_hbm.at[cid, pl.ds(sid * ROWS_PER_TILE, ROWS_PER_TILE)])


_agg_kernel = functools.partial(
    pl.kernel,
    out_type=jax.ShapeDtypeStruct((NC, NP, D), jnp.float32),
    mesh=_MESH,
    scratch_types=[
        pltpu.VMEM((CHUNK, D), jnp.float32),
        pltpu.VMEM((CHUNK, D), jnp.float32),
        pltpu.VMEM((CHUNK, D), jnp.float32),
        pltpu.VMEM((CHUNK,), jnp.int32),
        pltpu.VMEM((CHUNK,), jnp.int32),
        pltpu.VMEM((CHUNK,), jnp.int32),
        pltpu.VMEM((CHUNK,), jnp.int32),
        pltpu.VMEM((CHUNK,), jnp.int32),
        pltpu.VMEM((CHUNK,), jnp.int32),
        pltpu.VMEM_SHARED((NP, D), jnp.float32),
        pltpu.SemaphoreType.DMA,
        pltpu.SemaphoreType.DMA,
        pltpu.SemaphoreType.DMA,
        pltpu.SemaphoreType.DMA,
        pltpu.SemaphoreType.DMA,
        pltpu.SemaphoreType.DMA,
        pltpu.SemaphoreType.DMA,
        pltpu.SemaphoreType.DMA,
        pltpu.SemaphoreType.DMA,
    ],
)(_agg_body)


def _bn_scale_body(h_ref, gamma_ref, beta_ref, norm_ref, o_ref):
    h = h_ref[...]
    mean = jnp.mean(h, axis=0, keepdims=True)
    diff = h - mean
    var = jnp.mean(diff * diff, axis=0, keepdims=True)
    rstd = lax.rsqrt(var + 1e-5)
    o_ref[...] = (diff * (rstd * gamma_ref[...]) + beta_ref[...]) * norm_ref[...]


def _bn_scale(h, gamma, beta, norm_col):
    return pl.pallas_call(
        _bn_scale_body,
        out_shape=jax.ShapeDtypeStruct((N, D), jnp.float32),
    )(h, gamma.reshape(1, D), beta.reshape(1, D), norm_col)


def _mm_body(agg_ref, normin_ref, w_ref, b_ref, o_ref):
    m = (agg_ref[0, :N, :] + agg_ref[1, :N, :]) * normin_ref[...]
    mm = jnp.dot(m, w_ref[...], preferred_element_type=jnp.float32)
    o_ref[...] = jnp.maximum(mm + b_ref[...], 0.0)


def _mm_relu(agg, norm_in_col, w, b):
    return pl.pallas_call(
        _mm_body,
        out_shape=jax.ShapeDtypeStruct((N, D), jnp.float32),
    )(agg, norm_in_col, w, b.reshape(1, D))


def _mm_bn_body(agg_ref, normin_ref, w_ref, b_ref,
                gamma_ref, beta_ref, normout_ref, o_ref):
    m = (agg_ref[0, :N, :] + agg_ref[1, :N, :]) * normin_ref[...]
    mm = jnp.dot(m, w_ref[...], preferred_element_type=jnp.float32)
    h = jnp.maximum(mm + b_ref[...], 0.0)
    mean = jnp.mean(h, axis=0, keepdims=True)
    diff = h - mean
    var = jnp.mean(diff * diff, axis=0, keepdims=True)
    rstd = lax.rsqrt(var + 1e-5)
    o_ref[...] = (diff * (rstd * gamma_ref[...]) + beta_ref[...]) * normout_ref[...]


def _mm_relu_bn_scale(agg, norm_in_col, w, b, gamma, beta, norm_out_col):
    return pl.pallas_call(
        _mm_bn_body,
        out_shape=jax.ShapeDtypeStruct((N, D), jnp.float32),
    )(agg, norm_in_col, w, b.reshape(1, D),
      gamma.reshape(1, D), beta.reshape(1, D), norm_out_col)


def kernel(x, edge_index, gamma1, beta1, W1, b1, gamma2, beta2, W2, b2):
    src = edge_index[0].astype(jnp.int32)
    dst = edge_index[1].astype(jnp.int32)

    n_deg_pad = DEGPAD - 2 * E
    deg_pad = N + (jnp.arange(n_deg_pad, dtype=jnp.int32) % (DG - N))
    deg_idx = jnp.concatenate([src, dst + DG, deg_pad]).reshape(
        NW, DEG_CHUNKS, CHUNK)

    n_e_pad = EPAD - E
    src_pad = jnp.arange(n_e_pad, dtype=jnp.int32) % N
    dst_pad = N + (jnp.arange(n_e_pad, dtype=jnp.int32) % (NP - N))
    src_flat = jnp.concatenate([src, src_pad])
    dst_flat = jnp.concatenate([dst, dst_pad])

    deg_parts = _deg_kernel(deg_idx)
    deg = (deg_parts[0] + deg_parts[1]).reshape(2, DG)
    norm_out = lax.rsqrt(jnp.maximum(deg[0, :N], 1.0))[:, None]
    norm_in = lax.rsqrt(jnp.maximum(deg[1, :N], 1.0))[:, None]

    hs1 = _bn_scale(x, gamma1, beta1, norm_out)
    agg1 = _agg_kernel(hs1, src_flat, dst_flat)
    hs2 = _mm_relu_bn_scale(agg1, norm_in, W1, b1, gamma2, beta2, norm_out)
    agg2 = _agg_kernel(hs2, src_flat, dst_flat)
    h2 = _mm_relu(agg2, norm_in, W2, b2)
    return h2
